# baseline (speedup 1.0000x reference)
import functools

import jax
import jax.numpy as jnp
from jax import lax
from jax.experimental import pallas as pl
from jax.experimental.pallas import tpu as pltpu
from jax.experimental.pallas import tpu_sc as plsc

DIM = 32
NC = 2
NS = 16
NW = NC * NS
CHUNK = 1280
K = 1


@functools.lru_cache(maxsize=None)
def _make(B: int):
    n_per_w = B // NW
    n_chunks = n_per_w // CHUNK
    n_groups = n_chunks // K
    T = n_groups // 2
    assert n_per_w == n_chunks * CHUNK and n_chunks == n_groups * K
    assert n_groups == 2 * T
    mesh = plsc.VectorSubcoreMesh(core_axis_name="c", subcore_axis_name="s")

    @functools.partial(
        pl.kernel,
        mesh=mesh,
        compiler_params=pltpu.CompilerParams(use_tc_tiling_on_sc=False),
        out_type=jax.ShapeDtypeStruct((B, DIM), jnp.float32),
        scratch_types=[
            pltpu.VMEM((n_chunks, CHUNK), jnp.int32),
            pltpu.VMEM((K, CHUNK, DIM), jnp.float32),
            pltpu.VMEM((K, CHUNK, DIM), jnp.float32),
            pltpu.SemaphoreType.DMA,
            pltpu.SemaphoreType.DMA,
            pltpu.SemaphoreType.DMA,
            pltpu.SemaphoreType.DMA,
        ],
    )
    def gather_kernel(table_hbm, idx_hbm, out_hbm, idx_v, buf_a, buf_b,
                      gsem_a, gsem_b, osem_a, osem_b):
        wid = lax.axis_index("s") * NC + lax.axis_index("c")
        base = wid * n_per_w
        pltpu.sync_copy(idx_hbm.at[wid], idx_v)

        def fire_gathers(g, buf, sem):
            for i in range(K):
                pltpu.make_async_copy(
                    table_hbm.at[idx_v.at[g * K + i]], buf.at[i], sem).start()

        def drain(buf, sem):
            for i in range(K):
                pltpu.make_async_copy(
                    table_hbm.at[idx_v.at[i]], buf.at[i], sem).wait()

        def fire_outs(g, buf, sem):
            for i in range(K):
                pltpu.make_async_copy(
                    buf.at[i],
                    out_hbm.at[pl.ds(base + (g * K + i) * CHUNK, CHUNK)],
                    sem).start()

        fire_gathers(0, buf_a, gsem_a)

        def body(t, carry):
            g_a = 2 * t
            g_b = 2 * t + 1

            @pl.when(t > 0)
            def _():
                drain(buf_b, osem_b)

            fire_gathers(g_b, buf_b, gsem_b)

            drain(buf_a, gsem_a)
            fire_outs(g_a, buf_a, osem_a)

            @pl.when(t < T - 1)
            def _():
                drain(buf_a, osem_a)
                fire_gathers(g_a + 2, buf_a, gsem_a)

            drain(buf_b, gsem_b)
            fire_outs(g_b, buf_b, osem_b)
            return carry

        lax.fori_loop(0, T, body, 0)
        drain(buf_a, osem_a)
        drain(buf_b, osem_b)

    return gather_kernel


def kernel(input, weight):
    b, h = input.shape
    B = b * h
    idx3 = input.astype(jnp.int32).reshape(NW, B // NW // CHUNK, CHUNK)
    out = _make(B)(weight, idx3)
    return out.reshape(b, h, DIM)


# speedup vs baseline: 1.6273x; 1.6273x over previous
import functools

import jax
import jax.numpy as jnp
from jax import lax
from jax.experimental import pallas as pl
from jax.experimental.pallas import tpu as pltpu
from jax.experimental.pallas import tpu_sc as plsc

DIM = 32
NC = 2
NS = 16
NW = NC * NS
CHUNK = 1600
K = 1


@functools.lru_cache(maxsize=None)
def _make(b: int, h: int):
    B = b * h
    n_per_w = B // NW
    n_chunks = n_per_w // CHUNK
    n_groups = n_chunks // K
    T = n_groups // 2
    bc = CHUNK // h
    assert n_per_w == n_chunks * CHUNK and n_chunks == n_groups * K
    assert n_groups == 2 * T and bc * h == CHUNK and n_per_w % h == 0
    mesh = plsc.VectorSubcoreMesh(core_axis_name="c", subcore_axis_name="s")

    @functools.partial(
        pl.kernel,
        mesh=mesh,
        compiler_params=pltpu.CompilerParams(use_tc_tiling_on_sc=False),
        out_type=jax.ShapeDtypeStruct((b, h, DIM), jnp.float32),
        scratch_types=[
            pltpu.VMEM((n_chunks, CHUNK), jnp.int32),
            pltpu.VMEM((K, CHUNK, DIM), jnp.float32),
            pltpu.VMEM((K, CHUNK, DIM), jnp.float32),
            pltpu.SemaphoreType.DMA,
            pltpu.SemaphoreType.DMA,
            pltpu.SemaphoreType.DMA,
            pltpu.SemaphoreType.DMA,
        ],
    )
    def gather_kernel(table_hbm, idx_hbm, out_hbm, idx_v, buf_a, buf_b,
                      gsem_a, gsem_b, osem_a, osem_b):
        wid = lax.axis_index("s") * NC + lax.axis_index("c")
        base_b = wid * (n_per_w // h)
        pltpu.sync_copy(idx_hbm.at[wid], idx_v)

        def fire_gathers(g, buf, sem):
            for i in range(K):
                pltpu.make_async_copy(
                    table_hbm.at[idx_v.at[g * K + i]], buf.at[i], sem).start()

        def drain(buf, sem):
            for i in range(K):
                pltpu.make_async_copy(
                    table_hbm.at[idx_v.at[i]], buf.at[i], sem).wait()

        def fire_outs(g, buf, sem):
            for i in range(K):
                for k in range(bc):
                    pltpu.make_async_copy(
                        buf.at[i].at[pl.ds(k * h, h)],
                        out_hbm.at[base_b + (g * K + i) * bc + k],
                        sem).start()

        fire_gathers(0, buf_a, gsem_a)

        def body(t, carry):
            g_a = 2 * t
            g_b = 2 * t + 1

            @pl.when(t > 0)
            def _():
                drain(buf_b, osem_b)

            fire_gathers(g_b, buf_b, gsem_b)

            drain(buf_a, gsem_a)
            fire_outs(g_a, buf_a, osem_a)

            @pl.when(t < T - 1)
            def _():
                drain(buf_a, osem_a)
                fire_gathers(g_a + 2, buf_a, gsem_a)

            drain(buf_b, gsem_b)
            fire_outs(g_b, buf_b, osem_b)
            return carry

        lax.fori_loop(0, T, body, 0)
        drain(buf_a, osem_a)
        drain(buf_b, osem_b)

    return gather_kernel


def kernel(input, weight):
    b, h = input.shape
    B = b * h
    idx3 = input.astype(jnp.int32).reshape(NW, B // NW // CHUNK, CHUNK)
    return _make(b, h)(weight, idx3)


# speedup vs baseline: 1.6273x; 1.0000x over previous
import functools

import jax
import jax.numpy as jnp
from jax import lax
from jax.experimental import pallas as pl
from jax.experimental.pallas import tpu as pltpu
from jax.experimental.pallas import tpu_sc as plsc

DIM = 32
NC = 2
NS = 16
NW = NC * NS
CHUNK = 800
K = 2


@functools.lru_cache(maxsize=None)
def _make(b: int, h: int):
    B = b * h
    n_per_w = B // NW
    n_chunks = n_per_w // CHUNK
    n_groups = n_chunks // K
    T = n_groups // 2
    bc = CHUNK // h
    assert n_per_w == n_chunks * CHUNK and n_chunks == n_groups * K
    assert n_groups == 2 * T and bc * h == CHUNK and n_per_w % h == 0
    mesh = plsc.VectorSubcoreMesh(core_axis_name="c", subcore_axis_name="s")

    @functools.partial(
        pl.kernel,
        mesh=mesh,
        compiler_params=pltpu.CompilerParams(use_tc_tiling_on_sc=False),
        out_type=jax.ShapeDtypeStruct((b, h, DIM), jnp.float32),
        scratch_types=[
            pltpu.VMEM((n_chunks, CHUNK), jnp.int32),
            pltpu.VMEM((K, CHUNK, DIM), jnp.float32),
            pltpu.VMEM((K, CHUNK, DIM), jnp.float32),
            pltpu.SemaphoreType.DMA,
            pltpu.SemaphoreType.DMA,
            pltpu.SemaphoreType.DMA,
            pltpu.SemaphoreType.DMA,
        ],
    )
    def gather_kernel(table_hbm, idx_hbm, out_hbm, idx_v, buf_a, buf_b,
                      gsem_a, gsem_b, osem_a, osem_b):
        wid = lax.axis_index("s") * NC + lax.axis_index("c")
        base_b = wid * (n_per_w // h)
        pltpu.sync_copy(idx_hbm.at[wid], idx_v)

        def fire_gathers(g, buf, sem):
            for i in range(K):
                pltpu.make_async_copy(
                    table_hbm.at[idx_v.at[g * K + i]], buf.at[i], sem).start()

        def drain(buf, sem):
            for i in range(K):
                pltpu.make_async_copy(
                    table_hbm.at[idx_v.at[i]], buf.at[i], sem).wait()

        def fire_outs(g, buf, sem):
            for i in range(K):
                for k in range(bc):
                    pltpu.make_async_copy(
                        buf.at[i].at[pl.ds(k * h, h)],
                        out_hbm.at[base_b + (g * K + i) * bc + k],
                        sem).start()

        fire_gathers(0, buf_a, gsem_a)

        def body(t, carry):
            g_a = 2 * t
            g_b = 2 * t + 1

            @pl.when(t > 0)
            def _():
                drain(buf_b, osem_b)

            fire_gathers(g_b, buf_b, gsem_b)

            drain(buf_a, gsem_a)
            fire_outs(g_a, buf_a, osem_a)

            @pl.when(t < T - 1)
            def _():
                drain(buf_a, osem_a)
                fire_gathers(g_a + 2, buf_a, gsem_a)

            drain(buf_b, gsem_b)
            fire_outs(g_b, buf_b, osem_b)
            return carry

        lax.fori_loop(0, T, body, 0)
        drain(buf_a, osem_a)
        drain(buf_b, osem_b)

    return gather_kernel


def kernel(input, weight):
    b, h = input.shape
    B = b * h
    idx3 = input.astype(jnp.int32).reshape(NW, B // NW // CHUNK, CHUNK)
    return _make(b, h)(weight, idx3)
